# Initial kernel scaffold; baseline (speedup 1.0000x reference)
import jax
import jax.numpy as jnp
from jax.experimental import pallas as pl


def kernel(x, weights):
    raise NotImplementedError("write your pallas kernel here")


# speedup vs baseline: 1.6851x; 1.6851x over previous
import functools

import jax
import jax.numpy as jnp
from jax import lax
from jax.experimental import pallas as pl
from jax.experimental.pallas import tpu as pltpu
from jax.experimental.pallas import tpu_sc as plsc

_NC = 2
_NS = 16
_NW = _NC * _NS

_DIM = 64
_CHUNK = 128


@functools.partial(jax.jit, static_argnames=("n_chunks",))
def _sc_gather(idx, table, n_chunks):
    n_per_w = n_chunks * _CHUNK
    n_total = _NW * n_per_w
    mesh = plsc.VectorSubcoreMesh(core_axis_name="c", subcore_axis_name="s")

    @functools.partial(
        pl.kernel,
        out_type=jax.ShapeDtypeStruct((n_total, _DIM), jnp.float32),
        mesh=mesh,
        compiler_params=pltpu.CompilerParams(use_tc_tiling_on_sc=False),
        scratch_types=[
            pltpu.VMEM((n_chunks, _CHUNK), jnp.int32),
            pltpu.VMEM((_CHUNK, _DIM), jnp.float32),
            pltpu.SemaphoreType.DMA,
        ],
    )
    def k(idx_hbm, table_hbm, out_hbm, idx_v, rows_v, sem):
        wid = lax.axis_index("s") * _NC + lax.axis_index("c")
        base = wid * n_per_w
        pltpu.sync_copy(idx_hbm.at[wid], idx_v)

        def body(g, carry):
            pltpu.async_copy(table_hbm.at[idx_v.at[g]], rows_v, sem).wait()
            pltpu.sync_copy(
                rows_v, out_hbm.at[pl.ds(base + g * _CHUNK, _CHUNK)]
            )
            return carry

        lax.fori_loop(0, n_chunks, body, 0)

    return k(idx, table)


def kernel(x, weights):
    b, l = x.shape
    n = b * l
    n_chunks = n // (_NW * _CHUNK)
    idx = x.reshape(_NW, n_chunks, _CHUNK).astype(jnp.int32)
    out = _sc_gather(idx, weights, n_chunks)
    return out.reshape(b, l, weights.shape[1])


# speedup vs baseline: 1.8647x; 1.1065x over previous
import functools

import jax
import jax.numpy as jnp
from jax import lax
from jax.experimental import pallas as pl
from jax.experimental.pallas import tpu as pltpu
from jax.experimental.pallas import tpu_sc as plsc

_NC = 2
_NS = 16
_NW = _NC * _NS

_DIM = 64
_CHUNK = 128
_NBUF = 4


@functools.partial(jax.jit, static_argnames=("n_chunks",))
def _sc_gather(idx, table, n_chunks):
    n_per_w = n_chunks * _CHUNK
    n_total = _NW * n_per_w
    mesh = plsc.VectorSubcoreMesh(core_axis_name="c", subcore_axis_name="s")

    @functools.partial(
        pl.kernel,
        out_type=jax.ShapeDtypeStruct((n_total, _DIM), jnp.float32),
        mesh=mesh,
        compiler_params=pltpu.CompilerParams(use_tc_tiling_on_sc=False),
        scratch_types=[
            pltpu.VMEM((n_chunks, _CHUNK), jnp.int32),
            pltpu.VMEM((_NBUF, _CHUNK, _DIM), jnp.float32),
        ]
        + [pltpu.SemaphoreType.DMA] * (2 * _NBUF),
    )
    def k(idx_hbm, table_hbm, out_hbm, idx_v, rows_v, *sems):
        gsems = sems[:_NBUF]
        wsems = sems[_NBUF:]
        wid = lax.axis_index("s") * _NC + lax.axis_index("c")
        base = wid * n_per_w
        pltpu.sync_copy(idx_hbm.at[wid], idx_v)

        def out_at(g):
            return out_hbm.at[pl.ds(base + g * _CHUNK, _CHUNK)]

        pltpu.async_copy(table_hbm.at[idx_v.at[0]], rows_v.at[0], gsems[0])
        pltpu.async_copy(table_hbm.at[idx_v.at[1]], rows_v.at[1], gsems[1])

        def outer(j, carry):
            g0 = j * _NBUF
            for db in range(_NBUF):
                g = g0 + db
                pltpu.make_async_copy(
                    table_hbm.at[idx_v.at[g]], rows_v.at[db], gsems[db]
                ).wait()
                pltpu.async_copy(rows_v.at[db], out_at(g), wsems[db])
                nb = (db + 2) % _NBUF

                @pl.when(g + 2 < n_chunks)
                def _launch():
                    @pl.when(g >= 2)
                    def _drain_wb():
                        pltpu.make_async_copy(
                            rows_v.at[nb], out_at(0), wsems[nb]
                        ).wait()

                    pltpu.async_copy(
                        table_hbm.at[idx_v.at[g + 2]], rows_v.at[nb], gsems[nb]
                    )

            return carry

        lax.fori_loop(0, n_chunks // _NBUF, outer, 0)

        for db in range(_NBUF):
            pltpu.make_async_copy(rows_v.at[db], out_at(0), wsems[db]).wait()

    return k(idx, table)


def kernel(x, weights):
    b, l = x.shape
    n = b * l
    n_chunks = n // (_NW * _CHUNK)
    idx = x.reshape(_NW, n_chunks, _CHUNK).astype(jnp.int32)
    out = _sc_gather(idx, weights, n_chunks)
    return out.reshape(b, l, weights.shape[1])


# speedup vs baseline: 1.8785x; 1.0074x over previous
import functools

import jax
import jax.numpy as jnp
from jax import lax
from jax.experimental import pallas as pl
from jax.experimental.pallas import tpu as pltpu
from jax.experimental.pallas import tpu_sc as plsc

_NC = 2
_NS = 16
_NW = _NC * _NS

_DIM = 64
_CHUNK = 256
_NBUF = 4


@functools.partial(jax.jit, static_argnames=("n_chunks",))
def _sc_gather(idx, table, n_chunks):
    n_per_w = n_chunks * _CHUNK
    n_total = _NW * n_per_w
    mesh = plsc.VectorSubcoreMesh(core_axis_name="c", subcore_axis_name="s")

    @functools.partial(
        pl.kernel,
        out_type=jax.ShapeDtypeStruct((n_total, _DIM), jnp.float32),
        mesh=mesh,
        compiler_params=pltpu.CompilerParams(use_tc_tiling_on_sc=False),
        scratch_types=[
            pltpu.VMEM((n_chunks, _CHUNK), jnp.int32),
            pltpu.VMEM((_NBUF, _CHUNK, _DIM), jnp.float32),
        ]
        + [pltpu.SemaphoreType.DMA] * (2 * _NBUF),
    )
    def k(idx_hbm, table_hbm, out_hbm, idx_v, rows_v, *sems):
        gsems = sems[:_NBUF]
        wsems = sems[_NBUF:]
        wid = lax.axis_index("s") * _NC + lax.axis_index("c")
        base = wid * n_per_w
        pltpu.sync_copy(idx_hbm.at[wid], idx_v)

        def out_at(g):
            return out_hbm.at[pl.ds(base + g * _CHUNK, _CHUNK)]

        pltpu.async_copy(table_hbm.at[idx_v.at[0]], rows_v.at[0], gsems[0])
        pltpu.async_copy(table_hbm.at[idx_v.at[1]], rows_v.at[1], gsems[1])

        def outer(j, carry):
            g0 = j * _NBUF
            for db in range(_NBUF):
                g = g0 + db
                pltpu.make_async_copy(
                    table_hbm.at[idx_v.at[g]], rows_v.at[db], gsems[db]
                ).wait()
                pltpu.async_copy(rows_v.at[db], out_at(g), wsems[db])
                nb = (db + 2) % _NBUF

                @pl.when(g + 2 < n_chunks)
                def _launch():
                    @pl.when(g >= 2)
                    def _drain_wb():
                        pltpu.make_async_copy(
                            rows_v.at[nb], out_at(0), wsems[nb]
                        ).wait()

                    pltpu.async_copy(
                        table_hbm.at[idx_v.at[g + 2]], rows_v.at[nb], gsems[nb]
                    )

            return carry

        lax.fori_loop(0, n_chunks // _NBUF, outer, 0)

        for db in range(_NBUF):
            pltpu.make_async_copy(rows_v.at[db], out_at(0), wsems[db]).wait()

    return k(idx, table)


def kernel(x, weights):
    b, l = x.shape
    n = b * l
    n_chunks = n // (_NW * _CHUNK)
    idx = x.reshape(_NW, n_chunks, _CHUNK).astype(jnp.int32)
    out = _sc_gather(idx, weights, n_chunks)
    return out.reshape(b, l, weights.shape[1])


# speedup vs baseline: 1.8862x; 1.0041x over previous
import functools

import jax
import jax.numpy as jnp
from jax import lax
from jax.experimental import pallas as pl
from jax.experimental.pallas import tpu as pltpu
from jax.experimental.pallas import tpu_sc as plsc

_NC = 2
_NS = 16
_NW = _NC * _NS

_CB = 4
_NBUF = 4


@functools.partial(jax.jit, static_argnames=("b", "l", "d"))
def _sc_gather(idx, table, b, l, d):
    b_per_w = b // _NW
    n_chunks = b_per_w // _CB
    mesh = plsc.VectorSubcoreMesh(core_axis_name="c", subcore_axis_name="s")

    @functools.partial(
        pl.kernel,
        out_type=jax.ShapeDtypeStruct((b, l, d), jnp.float32),
        mesh=mesh,
        compiler_params=pltpu.CompilerParams(use_tc_tiling_on_sc=False),
        scratch_types=[
            pltpu.VMEM((b_per_w * l,), jnp.int32),
            pltpu.VMEM((_NBUF, _CB * l, d), jnp.float32),
        ]
        + [pltpu.SemaphoreType.DMA] * (2 * _NBUF),
    )
    def k(idx_hbm, table_hbm, out_hbm, idx_v, rows_v, *sems):
        gsems = sems[:_NBUF]
        wsems = sems[_NBUF:]
        wid = lax.axis_index("s") * _NC + lax.axis_index("c")
        base = wid * b_per_w
        pltpu.sync_copy(idx_hbm.at[pl.ds(base * l, b_per_w * l)], idx_v)

        def idx_at(g):
            return idx_v.at[pl.ds(g * _CB * l, _CB * l)]

        def wb_start(g, db):
            for r in range(_CB):
                pltpu.async_copy(
                    rows_v.at[db].at[pl.ds(r * l, l)],
                    out_hbm.at[base + g * _CB + r],
                    wsems[db],
                )

        def wb_wait(db):
            for r in range(_CB):
                pltpu.make_async_copy(
                    rows_v.at[db].at[pl.ds(r * l, l)],
                    out_hbm.at[0],
                    wsems[db],
                ).wait()

        pltpu.async_copy(table_hbm.at[idx_at(0)], rows_v.at[0], gsems[0])
        pltpu.async_copy(table_hbm.at[idx_at(1)], rows_v.at[1], gsems[1])

        def outer(j, carry):
            g0 = j * _NBUF
            for db in range(_NBUF):
                g = g0 + db
                pltpu.make_async_copy(
                    table_hbm.at[idx_at(g)], rows_v.at[db], gsems[db]
                ).wait()
                wb_start(g, db)
                nb = (db + 2) % _NBUF

                @pl.when(g + 2 < n_chunks)
                def _launch():
                    @pl.when(g >= 2)
                    def _drain_wb():
                        wb_wait(nb)

                    pltpu.async_copy(
                        table_hbm.at[idx_at(g + 2)], rows_v.at[nb], gsems[nb]
                    )

            return carry

        lax.fori_loop(0, n_chunks // _NBUF, outer, 0)

        for db in range(_NBUF):
            wb_wait(db)

    return k(idx, table)


def kernel(x, weights):
    b, l = x.shape
    idx = x.reshape(b * l).astype(jnp.int32)
    return _sc_gather(idx, weights, b, l, weights.shape[1])
